# baseline (speedup 1.0000x reference)
import jax
import jax.numpy as jnp
from jax.experimental import pallas as pl

N = 100000
BN = 2048


def _mlp_body(inp, w1, b1, w2, b2, w3, b3, w4, b4, w5, b5, out):
    h = jnp.maximum(jnp.dot(inp[...], w1[...], preferred_element_type=jnp.float32) + b1[...], 0.0)
    h = jnp.maximum(jnp.dot(h, w2[...], preferred_element_type=jnp.float32) + b2[...], 0.0)
    h = jnp.maximum(jnp.dot(h, w3[...], preferred_element_type=jnp.float32) + b3[...], 0.0)
    h = jnp.maximum(jnp.dot(h, w4[...], preferred_element_type=jnp.float32) + b4[...], 0.0)
    out[...] = jnp.dot(h, w5[...], preferred_element_type=jnp.float32) + b5[...]


def kernel(x, edge_index, edge_attr, u, batch, W1, b1, W2, b2, W3, b3, W4, b4, W5, b5):
    col = edge_index[1]
    dds = edge_attr[:, :2]
    hidden = edge_attr[:, 2:]
    dd = jax.ops.segment_sum(dds, col, num_segments=N)
    s = jax.ops.segment_sum(hidden, col, num_segments=N)
    mn = jax.ops.segment_min(hidden, col, num_segments=N)
    mx = jax.ops.segment_max(hidden, col, num_segments=N)
    cnt = jax.ops.segment_sum(jnp.ones((col.shape[0],), jnp.float32), col, num_segments=N)
    nonempty = (cnt > 0)[:, None]
    mn = jnp.where(nonempty, mn, 0.0)
    mx = jnp.where(nonempty, mx, 0.0)
    inputs = jnp.concatenate([dd, s, mn, mx], axis=1)

    npad = pl.cdiv(N, BN) * BN
    inputs = jnp.pad(inputs, ((0, npad - N), (0, 0)))
    grid = npad // BN
    out = pl.pallas_call(
        _mlp_body,
        grid=(grid,),
        in_specs=[
            pl.BlockSpec((BN, 17), lambda i: (i, 0)),
            pl.BlockSpec((17, 32), lambda i: (0, 0)),
            pl.BlockSpec((1, 32), lambda i: (0, 0)),
            pl.BlockSpec((32, 32), lambda i: (0, 0)),
            pl.BlockSpec((1, 32), lambda i: (0, 0)),
            pl.BlockSpec((32, 32), lambda i: (0, 0)),
            pl.BlockSpec((1, 32), lambda i: (0, 0)),
            pl.BlockSpec((32, 32), lambda i: (0, 0)),
            pl.BlockSpec((1, 32), lambda i: (0, 0)),
            pl.BlockSpec((32, 2), lambda i: (0, 0)),
            pl.BlockSpec((1, 2), lambda i: (0, 0)),
        ],
        out_specs=pl.BlockSpec((BN, 2), lambda i: (i, 0)),
        out_shape=jax.ShapeDtypeStruct((npad, 2), jnp.float32),
    )(inputs, W1, b1.reshape(1, 32), W2, b2.reshape(1, 32), W3, b3.reshape(1, 32),
      W4, b4.reshape(1, 32), W5, b5.reshape(1, 2))
    return out[:N]


# speedup vs baseline: 4.4254x; 4.4254x over previous
import functools

import jax
import jax.numpy as jnp
from jax import lax
from jax.experimental import pallas as pl
from jax.experimental.pallas import tpu as pltpu
from jax.experimental.pallas import tpu_sc as plsc

N = 100000
E = 3200000
NTILES = 32
W = 2048
CH0 = 100352
NF0 = CH0 // W
NF31 = (E - 31 * CH0) // W
TAIL31 = (E - 31 * CH0 - NF31 * W) // 128
BLK = 3200
NPAD = NTILES * BLK
ROWS = NPAD // 16
EB = E + NTILES * 16 + W
ACC = BLK * 5
MAXWIN = EB // W + 1

_mesh = plsc.VectorSubcoreMesh(core_axis_name="c", subcore_axis_name="s")


def _iota16():
    return lax.iota(jnp.int32, 16)


def _owner_local(cv):
    owner = lax.shift_right_logical(
        lax.shift_right_logical(cv, 7) * 164, 12)
    local = cv - owner * BLK
    return owner, local


def _fill_f32(ref, nelem, val):
    def b(i, _):
        ref[pl.ds(i * 16, 16)] = jnp.full((16,), val, jnp.float32)
        return 0
    lax.fori_loop(0, nelem // 16, b, 0)


def _fill_i32(ref, nelem, val):
    def b(i, _):
        ref[pl.ds(i * 16, 16)] = jnp.full((16,), val, jnp.int32)
        return 0
    lax.fori_loop(0, nelem // 16, b, 0)


def _k1_body(col2_hbm, attr_hbm, z8_hbm, o8_hbm,
             sums_hbm, hist_hbm,
             colbuf, abuf8, histbuf, acc8):
    c = lax.axis_index("c")
    s = lax.axis_index("s")
    w = c * 16 + s
    it16 = _iota16()
    onei = jnp.full((16,), 1, jnp.int32)

    _fill_i32(histbuf, 512, 0)
    pltpu.sync_copy(o8_hbm, abuf8)

    r0 = s * ROWS
    pltpu.sync_copy(z8_hbm.at[pl.ds(r0, ROWS), :], acc8.at[pl.ds(r0, ROWS), :])
    plsc.subcore_barrier()

    def window(row0, nsub):
        pltpu.sync_copy(col2_hbm.at[pl.ds(row0, nsub), :],
                        colbuf.at[pl.ds(0, nsub), :])
        pltpu.sync_copy(attr_hbm.at[pl.ds(row0 * 128, nsub * 128), :],
                        abuf8.at[pl.ds(0, nsub * 128), pl.ds(0, 7)])

        def fire(i, _):
            pltpu.sync_copy(abuf8.at[pl.ds(i * 128, 128), :],
                            acc8.at[colbuf.at[i]], add=True)
            return 0
        lax.fori_loop(0, nsub, fire, 0)

        def hist(k, _):
            cv = colbuf[k // 8, pl.ds((k % 8) * 16, 16)]
            owner, _ = _owner_local(cv)
            plsc.addupdate_scatter(histbuf, [owner * 16 + it16], onei)
            return 0
        lax.fori_loop(0, nsub * 8, hist, 0)

    def wloop(j, _):
        window(w * (CH0 // 128) + j * (W // 128), 16)
        return 0
    nfull = jnp.where(w == 31, NF31, NF0)
    lax.fori_loop(0, nfull, wloop, 0)

    @pl.when(w == 31)
    def _tail():
        window(31 * (CH0 // 128) + NF31 * (W // 128), TAIL31)

    plsc.subcore_barrier()
    pltpu.sync_copy(histbuf, hist_hbm.at[w])
    pltpu.sync_copy(acc8.at[pl.ds(r0, ROWS), :],
                    sums_hbm.at[c, pl.ds(r0, ROWS), :])


_k1 = functools.partial(
    pl.kernel,
    out_type=(
        jax.ShapeDtypeStruct((2, NPAD, 8), jnp.float32),
        jax.ShapeDtypeStruct((NTILES, 512), jnp.int32),
    ),
    mesh=_mesh,
    compiler_params=pltpu.CompilerParams(needs_layout_passes=False, use_tc_tiling_on_sc=False),
    scratch_types=[
        pltpu.VMEM((16, 128), jnp.int32),
        pltpu.VMEM((W, 8), jnp.float32),
        pltpu.VMEM((512,), jnp.int32),
        pltpu.VMEM_SHARED((NPAD, 8), jnp.float32),
    ],
)(_k1_body)


def _k2_body(col2_hbm, attr_hbm, base_hbm, loc_out, att_out,
             colbuf, attrbuf, posbuf, locbuf, cntbuf):
    c = lax.axis_index("c")
    s = lax.axis_index("s")
    w = c * 16 + s
    it16 = _iota16()

    pltpu.sync_copy(base_hbm.at[w], cntbuf)

    def window(row0, nsub):
        pltpu.sync_copy(col2_hbm.at[pl.ds(row0, nsub), :],
                        colbuf.at[pl.ds(0, nsub), :])
        pltpu.sync_copy(attr_hbm.at[pl.ds(row0 * 128, nsub * 128), :],
                        attrbuf.at[pl.ds(0, nsub * 128), pl.ds(0, 7)])

        def place(k, _):
            cv = colbuf[k // 8, pl.ds((k % 8) * 16, 16)]
            owner, local = _owner_local(cv)
            addr = owner * 16 + it16
            pos = plsc.load_gather(cntbuf, [addr])
            plsc.store_scatter(cntbuf, [addr], pos + 1)
            posbuf[k // 8, pl.ds((k % 8) * 16, 16)] = pos
            locbuf[k // 8, pl.ds((k % 8) * 16, 16)] = local
            return 0
        lax.fori_loop(0, nsub * 8, place, 0)

        def fire(i, _):
            pltpu.sync_copy(locbuf.at[i], loc_out.at[posbuf.at[i]])
            pltpu.sync_copy(attrbuf.at[pl.ds(i * 128, 128), :],
                            att_out.at[posbuf.at[i]])
            return 0
        lax.fori_loop(0, nsub, fire, 0)

    def wloop(j, _):
        window(w * (CH0 // 128) + j * (W // 128), 16)
        return 0
    nfull = jnp.where(w == 31, NF31, NF0)
    lax.fori_loop(0, nfull, wloop, 0)

    @pl.when(w == 31)
    def _tail():
        window(31 * (CH0 // 128) + NF31 * (W // 128), TAIL31)


_k2 = functools.partial(
    pl.kernel,
    out_type=(
        jax.ShapeDtypeStruct((EB,), jnp.int32),
        jax.ShapeDtypeStruct((EB, 8), jnp.float32),
    ),
    mesh=_mesh,
    compiler_params=pltpu.CompilerParams(needs_layout_passes=False, use_tc_tiling_on_sc=False),
    scratch_types=[
        pltpu.VMEM((16, 128), jnp.int32),
        pltpu.VMEM((W, 8), jnp.float32),
        pltpu.VMEM((16, 128), jnp.int32),
        pltpu.VMEM((16, 128), jnp.int32),
        pltpu.VMEM((512,), jnp.int32),
    ],
)(_k2_body)


def _k3_body(loc_hbm, att_hbm, segs_hbm, segc_hbm, min_out, max_out,
             locw, attw, accmin, accmax, segbuf):
    c = lax.axis_index("c")
    s = lax.axis_index("s")
    o = c * 16 + s
    it16 = _iota16()

    _fill_f32(accmin, ACC, jnp.inf)
    _fill_f32(accmax, ACC, -jnp.inf)

    pltpu.sync_copy(segs_hbm, segbuf.at[pl.ds(0, 32)])
    pltpu.sync_copy(segc_hbm, segbuf.at[pl.ds(32, 32)])
    sub = c * 16
    sv = segbuf[pl.ds(sub, 16)]
    nv = segbuf[pl.ds(32 + sub, 16)]
    pick = jnp.where(it16 == s, jnp.int32(1), jnp.int32(0))
    seg0 = pl.multiple_of(jnp.sum(sv * pick), 16)
    segn = jnp.sum(nv * pick)

    def vreg(k, joff):
        rem = segn - (joff + k * 16)
        lanemask = it16 < rem
        loc = locw[pl.ds(k * 16, 16)]
        loc5 = loc * 5
        rowi = k * 16 + it16
        vals = [plsc.load_gather(attw, [rowi, jnp.full((16,), cc + 2,
                                                       jnp.int32)])
                for cc in range(5)]

        def cond(m):
            return jnp.sum(jnp.where(m, jnp.int32(1), jnp.int32(0))) > 0

        def body(m):
            _, last = plsc.scan_count(loc, mask=m)
            for cc in range(5):
                a = loc5 + cc
                cur = plsc.load_gather(accmin, [a], mask=last)
                plsc.store_scatter(accmin, [a],
                                   jnp.minimum(cur, vals[cc]), mask=last)
                cur2 = plsc.load_gather(accmax, [a], mask=last)
                plsc.store_scatter(accmax, [a],
                                   jnp.maximum(cur2, vals[cc]), mask=last)
            return m & jnp.logical_not(last)

        lax.while_loop(cond, body, lanemask)
        return joff

    def wloop(j, _):
        @pl.when(j * W < segn)
        def _go():
            base = seg0 + j * W
            pltpu.sync_copy(loc_hbm.at[pl.ds(base, W)], locw)
            pltpu.sync_copy(att_hbm.at[pl.ds(base, W), :], attw)
            lax.fori_loop(0, W // 16, vreg, j * W)
        return 0
    lax.fori_loop(0, MAXWIN, wloop, 0)

    pltpu.sync_copy(accmin, min_out.at[o])
    pltpu.sync_copy(accmax, max_out.at[o])


_k3 = functools.partial(
    pl.kernel,
    out_type=(
        jax.ShapeDtypeStruct((NTILES, ACC), jnp.float32),
        jax.ShapeDtypeStruct((NTILES, ACC), jnp.float32),
    ),
    mesh=_mesh,
    compiler_params=pltpu.CompilerParams(needs_layout_passes=False, use_tc_tiling_on_sc=False),
    scratch_types=[
        pltpu.VMEM((W,), jnp.int32),
        pltpu.VMEM((W, 8), jnp.float32),
        pltpu.VMEM((ACC,), jnp.float32),
        pltpu.VMEM((ACC,), jnp.float32),
        pltpu.VMEM((64,), jnp.int32),
    ],
)(_k3_body)


BN = 2048


def _mlp_body(sp, mn, mx, w1, b1, w2, b2, w3, b3, w4, b4, w5, b5, out):
    sboth = sp[0] + sp[1]
    ssum = sboth[:, 0:7]
    ne = sboth[:, 7:8] > 0.0
    mnv = jnp.where(ne, mn[...], 0.0)
    mxv = jnp.where(ne, mx[...], 0.0)
    inp = jnp.concatenate([ssum, mnv, mxv], axis=1)
    h = jnp.maximum(jnp.dot(inp, w1[...], preferred_element_type=jnp.float32)
                    + b1[...], 0.0)
    h = jnp.maximum(jnp.dot(h, w2[...], preferred_element_type=jnp.float32)
                    + b2[...], 0.0)
    h = jnp.maximum(jnp.dot(h, w3[...], preferred_element_type=jnp.float32)
                    + b3[...], 0.0)
    h = jnp.maximum(jnp.dot(h, w4[...], preferred_element_type=jnp.float32)
                    + b4[...], 0.0)
    out[...] = (jnp.dot(h, w5[...], preferred_element_type=jnp.float32)
                + b5[...])


def kernel(x, edge_index, edge_attr, u, batch,
           W1, b1, W2, b2, W3, b3, W4, b4, W5, b5):
    col2 = edge_index[1].reshape(E // 128, 128)
    z8 = jnp.zeros((NPAD, 8), jnp.float32)
    o8 = jnp.ones((W, 8), jnp.float32)
    sums, hist = _k1(col2, edge_attr, z8, o8)

    h = hist.reshape(NTILES, 32, 16)
    tot = h.sum(axis=(0, 2))
    ptot = ((tot + 15) // 16) * 16
    segs = jnp.concatenate([jnp.zeros((1,), jnp.int32),
                            jnp.cumsum(ptot)[:-1].astype(jnp.int32)])
    how = h.transpose(1, 0, 2).reshape(32, NTILES * 16)
    ex = jnp.concatenate(
        [jnp.zeros((32, 1), jnp.int32),
         jnp.cumsum(how, axis=1)[:, :-1].astype(jnp.int32)], axis=1)
    base = (segs[:, None] + ex).reshape(32, NTILES, 16) \
        .transpose(1, 0, 2).reshape(NTILES, 512)

    loc, att = _k2(col2, edge_attr, base)
    mn, mx = _k3(loc, att, segs, tot.astype(jnp.int32))
    mn = mn.reshape(NPAD, 5)
    mx = mx.reshape(NPAD, 5)

    grid = NPAD // BN
    out = pl.pallas_call(
        _mlp_body,
        grid=(grid,),
        in_specs=[
            pl.BlockSpec((2, BN, 8), lambda i: (0, i, 0)),
            pl.BlockSpec((BN, 5), lambda i: (i, 0)),
            pl.BlockSpec((BN, 5), lambda i: (i, 0)),
            pl.BlockSpec((17, 32), lambda i: (0, 0)),
            pl.BlockSpec((1, 32), lambda i: (0, 0)),
            pl.BlockSpec((32, 32), lambda i: (0, 0)),
            pl.BlockSpec((1, 32), lambda i: (0, 0)),
            pl.BlockSpec((32, 32), lambda i: (0, 0)),
            pl.BlockSpec((1, 32), lambda i: (0, 0)),
            pl.BlockSpec((32, 32), lambda i: (0, 0)),
            pl.BlockSpec((1, 32), lambda i: (0, 0)),
            pl.BlockSpec((32, 2), lambda i: (0, 0)),
            pl.BlockSpec((1, 2), lambda i: (0, 0)),
        ],
        out_specs=pl.BlockSpec((BN, 2), lambda i: (i, 0)),
        out_shape=jax.ShapeDtypeStruct((NPAD, 2), jnp.float32),
    )(sums, mn, mx,
      W1, b1.reshape(1, 32), W2, b2.reshape(1, 32), W3, b3.reshape(1, 32),
      W4, b4.reshape(1, 32), W5, b5.reshape(1, 2))
    return out[:N]


# speedup vs baseline: 4.4305x; 1.0011x over previous
import functools

import jax
import jax.numpy as jnp
from jax import lax
from jax.experimental import pallas as pl
from jax.experimental.pallas import tpu as pltpu
from jax.experimental.pallas import tpu_sc as plsc

N = 100000
E = 3200000
NTILES = 32
W = 2048
CH0 = 100352
NF0 = CH0 // W
NF31 = (E - 31 * CH0) // W
TAIL31 = (E - 31 * CH0 - NF31 * W) // 128
BLK = 3200
NPAD = NTILES * BLK
ROWS = NPAD // 16
EB = E + NTILES * 16 + W
ACC = BLK * 5
MAXWIN = EB // W + 1

_mesh = plsc.VectorSubcoreMesh(core_axis_name="c", subcore_axis_name="s")


def _iota16():
    return lax.iota(jnp.int32, 16)


def _owner_local(cv):
    owner = lax.shift_right_logical(
        lax.shift_right_logical(cv, 7) * 164, 12)
    local = cv - owner * BLK
    return owner, local


def _fill_f32(ref, nelem, val):
    def b(i, _):
        ref[pl.ds(i * 16, 16)] = jnp.full((16,), val, jnp.float32)
        return 0
    lax.fori_loop(0, nelem // 16, b, 0)


def _fill_i32(ref, nelem, val):
    def b(i, _):
        ref[pl.ds(i * 16, 16)] = jnp.full((16,), val, jnp.int32)
        return 0
    lax.fori_loop(0, nelem // 16, b, 0)


def _k1_body(col2_hbm, attr_hbm, z8_hbm, o8_hbm,
             sums_hbm, hist_hbm,
             colbuf, abuf8, histbuf, acc8):
    c = lax.axis_index("c")
    s = lax.axis_index("s")
    w = c * 16 + s
    it16 = _iota16()
    onei = jnp.full((16,), 1, jnp.int32)

    _fill_i32(histbuf, 512, 0)
    pltpu.sync_copy(o8_hbm, abuf8)

    r0 = s * ROWS
    pltpu.sync_copy(z8_hbm.at[pl.ds(r0, ROWS), :], acc8.at[pl.ds(r0, ROWS), :])
    plsc.subcore_barrier()

    def window(row0, nsub):
        pltpu.sync_copy(col2_hbm.at[pl.ds(row0, nsub), :],
                        colbuf.at[pl.ds(0, nsub), :])
        pltpu.sync_copy(attr_hbm.at[pl.ds(row0 * 128, nsub * 128), :],
                        abuf8.at[pl.ds(0, nsub * 128), pl.ds(0, 7)])

        def fire(i, _):
            pltpu.sync_copy(abuf8.at[pl.ds(i * 128, 128), :],
                            acc8.at[colbuf.at[i]], add=True)
            return 0
        lax.fori_loop(0, nsub, fire, 0)

        def hist(k, _):
            cv = colbuf[k // 8, pl.ds((k % 8) * 16, 16)]
            owner, _ = _owner_local(cv)
            plsc.addupdate_scatter(histbuf, [owner * 16 + it16], onei)
            return 0
        lax.fori_loop(0, nsub * 8, hist, 0)

    def wloop(j, _):
        window(w * (CH0 // 128) + j * (W // 128), 16)
        return 0
    nfull = jnp.where(w == 31, NF31, NF0)
    lax.fori_loop(0, nfull, wloop, 0)

    @pl.when(w == 31)
    def _tail():
        window(31 * (CH0 // 128) + NF31 * (W // 128), TAIL31)

    plsc.subcore_barrier()
    pltpu.sync_copy(histbuf, hist_hbm.at[w])
    pltpu.sync_copy(acc8.at[pl.ds(r0, ROWS), :],
                    sums_hbm.at[c, pl.ds(r0, ROWS), :])


_k1 = functools.partial(
    pl.kernel,
    out_type=(
        jax.ShapeDtypeStruct((2, NPAD, 8), jnp.float32),
        jax.ShapeDtypeStruct((NTILES, 512), jnp.int32),
    ),
    mesh=_mesh,
    compiler_params=pltpu.CompilerParams(needs_layout_passes=False, use_tc_tiling_on_sc=False),
    scratch_types=[
        pltpu.VMEM((16, 128), jnp.int32),
        pltpu.VMEM((W, 8), jnp.float32),
        pltpu.VMEM((512,), jnp.int32),
        pltpu.VMEM_SHARED((NPAD, 8), jnp.float32),
    ],
)(_k1_body)


def _k2_body(col2_hbm, attr_hbm, base_hbm, loc_out, att_out,
             colbuf, attrbuf, posbuf, locbuf, cntbuf, semL, semA):
    c = lax.axis_index("c")
    s = lax.axis_index("s")
    w = c * 16 + s
    it16 = _iota16()

    pltpu.sync_copy(base_hbm.at[w], cntbuf)

    def window(row0, nsub):
        pltpu.sync_copy(col2_hbm.at[pl.ds(row0, nsub), :],
                        colbuf.at[pl.ds(0, nsub), :])
        pltpu.sync_copy(attr_hbm.at[pl.ds(row0 * 128, nsub * 128), :],
                        attrbuf.at[pl.ds(0, nsub * 128), pl.ds(0, 7)])

        def place(k, _):
            cv = colbuf[k // 8, pl.ds((k % 8) * 16, 16)]
            owner, local = _owner_local(cv)
            addr = owner * 16 + it16
            pos = plsc.load_gather(cntbuf, [addr])
            plsc.store_scatter(cntbuf, [addr], pos + 1)
            posbuf[k // 8, pl.ds((k % 8) * 16, 16)] = pos
            locbuf[k // 8, pl.ds((k % 8) * 16, 16)] = local
            return 0
        lax.fori_loop(0, nsub * 8, place, 0)

        def fire(i, _):
            d1 = pltpu.async_copy(locbuf.at[i], loc_out.at[posbuf.at[i]],
                                  semL)
            d2 = pltpu.async_copy(attrbuf.at[pl.ds(i * 128, 128), :],
                                  att_out.at[posbuf.at[i]], semA)
            d1.wait()
            d2.wait()
            return 0
        lax.fori_loop(0, nsub, fire, 0)

    def wloop(j, _):
        window(w * (CH0 // 128) + j * (W // 128), 16)
        return 0
    nfull = jnp.where(w == 31, NF31, NF0)
    lax.fori_loop(0, nfull, wloop, 0)

    @pl.when(w == 31)
    def _tail():
        window(31 * (CH0 // 128) + NF31 * (W // 128), TAIL31)


_k2 = functools.partial(
    pl.kernel,
    out_type=(
        jax.ShapeDtypeStruct((EB,), jnp.int32),
        jax.ShapeDtypeStruct((EB, 8), jnp.float32),
    ),
    mesh=_mesh,
    compiler_params=pltpu.CompilerParams(needs_layout_passes=False, use_tc_tiling_on_sc=False),
    scratch_types=[
        pltpu.VMEM((16, 128), jnp.int32),
        pltpu.VMEM((W, 8), jnp.float32),
        pltpu.VMEM((16, 128), jnp.int32),
        pltpu.VMEM((16, 128), jnp.int32),
        pltpu.VMEM((512,), jnp.int32),
        pltpu.SemaphoreType.DMA,
        pltpu.SemaphoreType.DMA,
    ],
)(_k2_body)


def _k3_body(loc_hbm, att_hbm, segs_hbm, segc_hbm, min_out, max_out,
             locw, attw, accmin, accmax, segbuf):
    c = lax.axis_index("c")
    s = lax.axis_index("s")
    o = c * 16 + s
    it16 = _iota16()

    _fill_f32(accmin, ACC, jnp.inf)
    _fill_f32(accmax, ACC, -jnp.inf)

    pltpu.sync_copy(segs_hbm, segbuf.at[pl.ds(0, 32)])
    pltpu.sync_copy(segc_hbm, segbuf.at[pl.ds(32, 32)])
    sub = c * 16
    sv = segbuf[pl.ds(sub, 16)]
    nv = segbuf[pl.ds(32 + sub, 16)]
    pick = jnp.where(it16 == s, jnp.int32(1), jnp.int32(0))
    seg0 = pl.multiple_of(jnp.sum(sv * pick), 16)
    segn = jnp.sum(nv * pick)

    def vreg(k, joff):
        rem = segn - (joff + k * 16)
        lanemask = it16 < rem
        loc = locw[pl.ds(k * 16, 16)]
        loc5 = loc * 5
        rowi = k * 16 + it16
        vals = [plsc.load_gather(attw, [rowi, jnp.full((16,), cc + 2,
                                                       jnp.int32)])
                for cc in range(5)]

        def cond(m):
            return jnp.sum(jnp.where(m, jnp.int32(1), jnp.int32(0))) > 0

        def body(m):
            _, last = plsc.scan_count(loc, mask=m)
            for cc in range(5):
                a = loc5 + cc
                cur = plsc.load_gather(accmin, [a], mask=last)
                plsc.store_scatter(accmin, [a],
                                   jnp.minimum(cur, vals[cc]), mask=last)
                cur2 = plsc.load_gather(accmax, [a], mask=last)
                plsc.store_scatter(accmax, [a],
                                   jnp.maximum(cur2, vals[cc]), mask=last)
            return m & jnp.logical_not(last)

        lax.while_loop(cond, body, lanemask)
        return joff

    def wloop(j, _):
        @pl.when(j * W < segn)
        def _go():
            base = seg0 + j * W
            pltpu.sync_copy(loc_hbm.at[pl.ds(base, W)], locw)
            pltpu.sync_copy(att_hbm.at[pl.ds(base, W), :], attw)
            lax.fori_loop(0, W // 16, vreg, j * W)
        return 0
    lax.fori_loop(0, MAXWIN, wloop, 0)

    pltpu.sync_copy(accmin, min_out.at[o])
    pltpu.sync_copy(accmax, max_out.at[o])


_k3 = functools.partial(
    pl.kernel,
    out_type=(
        jax.ShapeDtypeStruct((NTILES, ACC), jnp.float32),
        jax.ShapeDtypeStruct((NTILES, ACC), jnp.float32),
    ),
    mesh=_mesh,
    compiler_params=pltpu.CompilerParams(needs_layout_passes=False, use_tc_tiling_on_sc=False),
    scratch_types=[
        pltpu.VMEM((W,), jnp.int32),
        pltpu.VMEM((W, 8), jnp.float32),
        pltpu.VMEM((ACC,), jnp.float32),
        pltpu.VMEM((ACC,), jnp.float32),
        pltpu.VMEM((64,), jnp.int32),
    ],
)(_k3_body)


BN = 2048


def _mlp_body(sp, mn, mx, w1, b1, w2, b2, w3, b3, w4, b4, w5, b5, out):
    sboth = sp[0] + sp[1]
    ssum = sboth[:, 0:7]
    ne = sboth[:, 7:8] > 0.0
    mnv = jnp.where(ne, mn[...], 0.0)
    mxv = jnp.where(ne, mx[...], 0.0)
    inp = jnp.concatenate([ssum, mnv, mxv], axis=1)
    h = jnp.maximum(jnp.dot(inp, w1[...], preferred_element_type=jnp.float32)
                    + b1[...], 0.0)
    h = jnp.maximum(jnp.dot(h, w2[...], preferred_element_type=jnp.float32)
                    + b2[...], 0.0)
    h = jnp.maximum(jnp.dot(h, w3[...], preferred_element_type=jnp.float32)
                    + b3[...], 0.0)
    h = jnp.maximum(jnp.dot(h, w4[...], preferred_element_type=jnp.float32)
                    + b4[...], 0.0)
    out[...] = (jnp.dot(h, w5[...], preferred_element_type=jnp.float32)
                + b5[...])


def kernel(x, edge_index, edge_attr, u, batch,
           W1, b1, W2, b2, W3, b3, W4, b4, W5, b5):
    col2 = edge_index[1].reshape(E // 128, 128)
    z8 = jnp.zeros((NPAD, 8), jnp.float32)
    o8 = jnp.ones((W, 8), jnp.float32)
    sums, hist = _k1(col2, edge_attr, z8, o8)

    h = hist.reshape(NTILES, 32, 16)
    tot = h.sum(axis=(0, 2))
    ptot = ((tot + 15) // 16) * 16
    segs = jnp.concatenate([jnp.zeros((1,), jnp.int32),
                            jnp.cumsum(ptot)[:-1].astype(jnp.int32)])
    how = h.transpose(1, 0, 2).reshape(32, NTILES * 16)
    ex = jnp.concatenate(
        [jnp.zeros((32, 1), jnp.int32),
         jnp.cumsum(how, axis=1)[:, :-1].astype(jnp.int32)], axis=1)
    base = (segs[:, None] + ex).reshape(32, NTILES, 16) \
        .transpose(1, 0, 2).reshape(NTILES, 512)

    loc, att = _k2(col2, edge_attr, base)
    mn, mx = _k3(loc, att, segs, tot.astype(jnp.int32))
    mn = mn.reshape(NPAD, 5)
    mx = mx.reshape(NPAD, 5)

    grid = NPAD // BN
    out = pl.pallas_call(
        _mlp_body,
        grid=(grid,),
        in_specs=[
            pl.BlockSpec((2, BN, 8), lambda i: (0, i, 0)),
            pl.BlockSpec((BN, 5), lambda i: (i, 0)),
            pl.BlockSpec((BN, 5), lambda i: (i, 0)),
            pl.BlockSpec((17, 32), lambda i: (0, 0)),
            pl.BlockSpec((1, 32), lambda i: (0, 0)),
            pl.BlockSpec((32, 32), lambda i: (0, 0)),
            pl.BlockSpec((1, 32), lambda i: (0, 0)),
            pl.BlockSpec((32, 32), lambda i: (0, 0)),
            pl.BlockSpec((1, 32), lambda i: (0, 0)),
            pl.BlockSpec((32, 32), lambda i: (0, 0)),
            pl.BlockSpec((1, 32), lambda i: (0, 0)),
            pl.BlockSpec((32, 2), lambda i: (0, 0)),
            pl.BlockSpec((1, 2), lambda i: (0, 0)),
        ],
        out_specs=pl.BlockSpec((BN, 2), lambda i: (i, 0)),
        out_shape=jax.ShapeDtypeStruct((NPAD, 2), jnp.float32),
    )(sums, mn, mx,
      W1, b1.reshape(1, 32), W2, b2.reshape(1, 32), W3, b3.reshape(1, 32),
      W4, b4.reshape(1, 32), W5, b5.reshape(1, 2))
    return out[:N]
